# baseline (speedup 1.0000x reference)
import functools

import jax
import jax.numpy as jnp
from jax import lax
from jax.experimental import pallas as pl
from jax.experimental.pallas import tpu as pltpu

_LN2 = 0.6931471805599453


def _ssp(v):
    return jnp.maximum(v, 0.0) + jnp.log1p(jnp.exp(-jnp.abs(v))) - _LN2


def _yt_body(x_ref, w_ref, yt_ref):
    m = jnp.dot(x_ref[0].astype(jnp.bfloat16), w_ref[...],
                preferred_element_type=jnp.float32)
    yt_ref[0] = m.T.astype(jnp.bfloat16)


def _main_body(NT, NBR, AT, f_ref, nb_ref, mk_ref, x_ref, yt_ref,
               wf1t_ref, bf1_ref, wf2t_ref, bf2_ref, wfo_ref, bfo_ref,
               wd_ref, bd_ref, o_ref, acc_ref):
    t = pl.program_id(1)
    fv = f_ref[0]
    nbv = nb_ref[0]
    mkv = mk_ref[0]
    ytab = yt_ref[0]
    rows = lax.broadcasted_iota(jnp.int16, (AT, AT), 0)
    zsum = None
    for n in range(NT):
        fn = fv[n].astype(jnp.bfloat16)
        ht = jnp.dot(wf1t_ref[...], fn,
                     preferred_element_type=jnp.float32).astype(jnp.bfloat16)
        ht = _ssp(ht + bf1_ref[...])
        wt = jnp.dot(wf2t_ref[...], ht,
                     preferred_element_type=jnp.float32) + bf2_ref[...]
        nbn = nbv[n].astype(jnp.int16)[None, :]
        mkn = mkv[n].astype(jnp.bfloat16)[None, :]
        onehot = jnp.where(rows == nbn, mkn, jnp.bfloat16(0.0))
        ynbr = jnp.dot(ytab, onehot, preferred_element_type=jnp.float32)
        z = wt * ynbr
        zsum = z if zsum is None else zsum + z
    @pl.when(t == 0)
    def _init():
        acc_ref[...] = zsum

    @pl.when(t != 0)
    def _accum():
        acc_ref[...] += zsum

    @pl.when(t == NBR // NT - 1)
    def _head():
        agg = acc_ref[...].T
        v = _ssp(jnp.dot(agg.astype(jnp.bfloat16), wfo_ref[...],
                         preferred_element_type=jnp.float32) + bfo_ref[...])
        out = jnp.dot(v.astype(jnp.bfloat16), wd_ref[...],
                      preferred_element_type=jnp.float32)
        o_ref[0] = out + bd_ref[...] + x_ref[0]


def kernel(x, f_double, neighbors, neighbor_mask, Wf1, bf1, Wf2, bf2,
           W_in2f, W_f2out, b_f2out, W_dense, b_dense):
    B, AT, NBR = neighbors.shape
    G = f_double.shape[-1]
    F = Wf1.shape[1]
    NAB = x.shape[-1]
    NT = 8
    nS = NBR // NT

    yt = pl.pallas_call(
        _yt_body,
        grid=(B,),
        in_specs=[
            pl.BlockSpec((1, AT, NAB), lambda b: (b, 0, 0)),
            pl.BlockSpec((NAB, F), lambda b: (0, 0)),
        ],
        out_specs=pl.BlockSpec((1, F, AT), lambda b: (b, 0, 0)),
        out_shape=jax.ShapeDtypeStruct((B, F, AT), jnp.bfloat16),
    )(x, W_in2f.astype(jnp.bfloat16))

    ft = jnp.transpose(f_double, (0, 2, 3, 1))
    nbt = jnp.transpose(neighbors, (0, 2, 1))
    mkt = jnp.transpose(neighbor_mask, (0, 2, 1))

    out = pl.pallas_call(
        functools.partial(_main_body, NT, NBR, AT),
        grid=(B, nS),
        in_specs=[
            pl.BlockSpec((1, NT, G, AT), lambda b, t: (b, t, 0, 0)),
            pl.BlockSpec((1, NT, AT), lambda b, t: (b, t, 0)),
            pl.BlockSpec((1, NT, AT), lambda b, t: (b, t, 0)),
            pl.BlockSpec((1, AT, NAB), lambda b, t: (b, 0, 0)),
            pl.BlockSpec((1, F, AT), lambda b, t: (b, 0, 0)),
            pl.BlockSpec((F, G), lambda b, t: (0, 0)),
            pl.BlockSpec((F, 1), lambda b, t: (0, 0)),
            pl.BlockSpec((F, F), lambda b, t: (0, 0)),
            pl.BlockSpec((F, 1), lambda b, t: (0, 0)),
            pl.BlockSpec((F, NAB), lambda b, t: (0, 0)),
            pl.BlockSpec((1, NAB), lambda b, t: (0, 0)),
            pl.BlockSpec((NAB, NAB), lambda b, t: (0, 0)),
            pl.BlockSpec((1, NAB), lambda b, t: (0, 0)),
        ],
        out_specs=pl.BlockSpec((1, AT, NAB), lambda b, t: (b, 0, 0)),
        out_shape=jax.ShapeDtypeStruct((B, AT, NAB), jnp.float32),
        scratch_shapes=[pltpu.VMEM((F, AT), jnp.float32)],
    )(ft, nbt, mkt, x, yt,
      Wf1.T.astype(jnp.bfloat16), bf1.astype(jnp.bfloat16).reshape(F, 1),
      Wf2.T.astype(jnp.bfloat16), bf2.reshape(F, 1),
      W_f2out.astype(jnp.bfloat16), b_f2out.reshape(1, NAB),
      W_dense.astype(jnp.bfloat16), b_dense.reshape(1, NAB))
    return out


# speedup vs baseline: 1.2333x; 1.2333x over previous
import functools

import jax
import jax.numpy as jnp
from jax import lax
from jax.experimental import pallas as pl

_LN2 = 0.6931471805599453


def _ssp(v):
    return jnp.log(jnp.exp(v) + 1.0) - _LN2


def _y_body(x_ref, w_ref, y_ref):
    y_ref[...] = jnp.dot(
        x_ref[...].astype(jnp.bfloat16), w_ref[...],
        preferred_element_type=jnp.float32,
    ).astype(jnp.bfloat16)


def _main_body(A_T, NBR, AT, f_ref, nb_ref, mk_ref, x_ref, y_ref,
               wf1_ref, bf1_ref, wf2_ref, bf2_ref, wfo_ref, bfo_ref,
               wd_ref, bd_ref, o_ref):
    E = A_T * NBR
    F = wf1_ref.shape[1]
    f = f_ref[0].reshape(E, -1).astype(jnp.bfloat16)
    h = jnp.dot(f, wf1_ref[...],
                preferred_element_type=jnp.float32).astype(jnp.bfloat16)
    h = _ssp(h + bf1_ref[...])
    w = jnp.dot(h, wf2_ref[...], preferred_element_type=jnp.float32)
    w = w + bf2_ref[...]
    nb = nb_ref[0].astype(jnp.int16)
    mk = mk_ref[0].astype(jnp.bfloat16)
    onehot = jnp.where(
        nb[:, :, None] == lax.broadcasted_iota(
            jnp.int16, (A_T, NBR, AT), 2),
        mk[:, :, None], jnp.bfloat16(0.0)).reshape(E, AT)
    ynbr = jnp.dot(onehot, y_ref[0], preferred_element_type=jnp.float32)
    z = w * ynbr
    agg = z.reshape(A_T, NBR, F).sum(axis=1)
    v = _ssp(jnp.dot(agg.astype(jnp.bfloat16), wfo_ref[...],
                     preferred_element_type=jnp.float32) + bfo_ref[...])
    out = jnp.dot(v.astype(jnp.bfloat16), wd_ref[...],
                  preferred_element_type=jnp.float32)
    o_ref[0] = out + bd_ref[...] + x_ref[0]


def kernel(x, f_double, neighbors, neighbor_mask, Wf1, bf1, Wf2, bf2,
           W_in2f, W_f2out, b_f2out, W_dense, b_dense):
    B, AT, NBR = neighbors.shape
    G = f_double.shape[-1]
    F = Wf1.shape[1]
    NAB = x.shape[-1]
    A_T = 32
    nT = AT // A_T

    x2 = x.reshape(B * AT, NAB)
    YR = 4
    y = pl.pallas_call(
        _y_body,
        grid=(YR,),
        in_specs=[
            pl.BlockSpec((B * AT // YR, NAB), lambda i: (i, 0)),
            pl.BlockSpec((NAB, F), lambda i: (0, 0)),
        ],
        out_specs=pl.BlockSpec((B * AT // YR, F), lambda i: (i, 0)),
        out_shape=jax.ShapeDtypeStruct((B * AT, F), jnp.bfloat16),
    )(x2, W_in2f.astype(jnp.bfloat16))
    y3 = y.reshape(B, AT, F)

    out = pl.pallas_call(
        functools.partial(_main_body, A_T, NBR, AT),
        grid=(B, nT),
        in_specs=[
            pl.BlockSpec((1, A_T, NBR, G), lambda b, t: (b, t, 0, 0)),
            pl.BlockSpec((1, A_T, NBR), lambda b, t: (b, t, 0)),
            pl.BlockSpec((1, A_T, NBR), lambda b, t: (b, t, 0)),
            pl.BlockSpec((1, A_T, NAB), lambda b, t: (b, t, 0)),
            pl.BlockSpec((1, AT, F), lambda b, t: (b, 0, 0)),
            pl.BlockSpec((G, F), lambda b, t: (0, 0)),
            pl.BlockSpec((1, F), lambda b, t: (0, 0)),
            pl.BlockSpec((F, F), lambda b, t: (0, 0)),
            pl.BlockSpec((1, F), lambda b, t: (0, 0)),
            pl.BlockSpec((F, NAB), lambda b, t: (0, 0)),
            pl.BlockSpec((1, NAB), lambda b, t: (0, 0)),
            pl.BlockSpec((NAB, NAB), lambda b, t: (0, 0)),
            pl.BlockSpec((1, NAB), lambda b, t: (0, 0)),
        ],
        out_specs=pl.BlockSpec((1, A_T, NAB), lambda b, t: (b, t, 0)),
        out_shape=jax.ShapeDtypeStruct((B, AT, NAB), jnp.float32),
    )(f_double, neighbors, neighbor_mask, x, y3,
      Wf1.astype(jnp.bfloat16), bf1.astype(jnp.bfloat16).reshape(1, F),
      Wf2.astype(jnp.bfloat16), bf2.reshape(1, F),
      W_f2out.astype(jnp.bfloat16), b_f2out.reshape(1, NAB),
      W_dense.astype(jnp.bfloat16), b_dense.reshape(1, NAB))
    return out
